# Initial kernel scaffold; baseline (speedup 1.0000x reference)
import jax
import jax.numpy as jnp
from jax.experimental import pallas as pl


def kernel(x, table):
    raise NotImplementedError("write your pallas kernel here")


# speedup vs baseline: 1.1024x; 1.1024x over previous
import functools

import jax
import jax.numpy as jnp
from jax import lax
from jax.experimental import pallas as pl
from jax.experimental.pallas import tpu as pltpu
from jax.experimental.pallas import tpu_sc as plsc

VOCAB = 1000000
DIM = 32

_info = plsc.get_sparse_core_info()
_NC, _NS = _info.num_cores, _info.num_subcores
_NW = _NC * _NS

_B = 16384 * 50
_BPW = _B // _NW
_CHUNK = 1600
_NCHUNK = _BPW // _CHUNK


@functools.partial(
    pl.kernel,
    mesh=plsc.VectorSubcoreMesh(core_axis_name="c", subcore_axis_name="s"),
    out_type=jax.ShapeDtypeStruct((_B, DIM), jnp.float32),
    scratch_types=[
        pltpu.VMEM((_CHUNK,), jnp.int32),
        pltpu.VMEM((_CHUNK, DIM), jnp.float32),
        pltpu.SemaphoreType.DMA,
    ],
    compiler_params=pltpu.CompilerParams(use_tc_tiling_on_sc=False),
)
def _gather_kernel(idx_hbm, table_hbm, out_hbm, idx_v, rows_v, sem):
    wid = lax.axis_index("s") * _NC + lax.axis_index("c")
    base = wid * _BPW
    for i in range(_NCHUNK):
        off = base + i * _CHUNK
        pltpu.sync_copy(idx_hbm.at[pl.ds(off, _CHUNK)], idx_v)
        pltpu.async_copy(table_hbm.at[idx_v], rows_v, sem).wait()
        pltpu.sync_copy(rows_v, out_hbm.at[pl.ds(off, _CHUNK)])


def kernel(x, table):
    n, s = x.shape
    flat = x.reshape(n * s).astype(jnp.int32)
    out = _gather_kernel(flat, table)
    return out.reshape(n, s, DIM)


# speedup vs baseline: 1.1138x; 1.0104x over previous
import functools

import jax
import jax.numpy as jnp
from jax import lax
from jax.experimental import pallas as pl
from jax.experimental.pallas import tpu as pltpu
from jax.experimental.pallas import tpu_sc as plsc

VOCAB = 1000000
DIM = 32

_info = plsc.get_sparse_core_info()
_NC, _NS = _info.num_cores, _info.num_subcores
_NW = _NC * _NS

_B = 16384 * 50
_BPW = _B // _NW
_CHUNK = 1600
_NCHUNK = _BPW // _CHUNK


@functools.partial(
    pl.kernel,
    mesh=plsc.VectorSubcoreMesh(core_axis_name="c", subcore_axis_name="s"),
    out_type=jax.ShapeDtypeStruct((_B, DIM), jnp.float32),
    scratch_types=[
        pltpu.VMEM((_BPW,), jnp.int32),
        pltpu.VMEM((_CHUNK, DIM), jnp.float32),
        pltpu.VMEM((_CHUNK, DIM), jnp.float32),
        pltpu.SemaphoreType.DMA,
        pltpu.SemaphoreType.DMA,
        pltpu.SemaphoreType.DMA,
        pltpu.SemaphoreType.DMA,
    ],
    compiler_params=pltpu.CompilerParams(use_tc_tiling_on_sc=False),
)
def _gather_kernel(idx_hbm, table_hbm, out_hbm, idx_all, rows0, rows1,
                   gsem0, gsem1, ssem0, ssem1):
    wid = lax.axis_index("s") * _NC + lax.axis_index("c")
    base = wid * _BPW
    rows = (rows0, rows1)
    gsems = (gsem0, gsem1)
    ssems = (ssem0, ssem1)

    pltpu.sync_copy(idx_hbm.at[pl.ds(base, _BPW)], idx_all)

    gathers = [None] * _NCHUNK
    stores = [None] * _NCHUNK
    for i in range(_NCHUNK):
        s = i % 2
        if i >= 2:
            stores[i - 2].wait()
        gathers[i] = pltpu.async_copy(
            table_hbm.at[idx_all.at[pl.ds(i * _CHUNK, _CHUNK)]],
            rows[s], gsems[s])
        if i >= 1:
            p = 1 - s
            gathers[i - 1].wait()
            stores[i - 1] = pltpu.async_copy(
                rows[p], out_hbm.at[pl.ds(base + (i - 1) * _CHUNK, _CHUNK)],
                ssems[p])
    last = _NCHUNK - 1
    gathers[last].wait()
    stores[last] = pltpu.async_copy(
        rows[last % 2], out_hbm.at[pl.ds(base + last * _CHUNK, _CHUNK)],
        ssems[last % 2])
    stores[last - 1].wait()
    stores[last].wait()


def kernel(x, table):
    n, s = x.shape
    flat = x.reshape(n * s).astype(jnp.int32)
    out = _gather_kernel(flat, table)
    return out.reshape(n, s, DIM)


# speedup vs baseline: 1.7948x; 1.6114x over previous
import functools

import jax
import jax.numpy as jnp
from jax import lax
from jax.experimental import pallas as pl
from jax.experimental.pallas import tpu as pltpu
from jax.experimental.pallas import tpu_sc as plsc

VOCAB = 1000000
DIM = 32
SEQ = 50
BATCH = 16384

_info = plsc.get_sparse_core_info()
_NC, _NS = _info.num_cores, _info.num_subcores
_NW = _NC * _NS
_IPW = BATCH // _NW
_ICHUNK = 32
_NCHUNK = _IPW // _ICHUNK


@functools.partial(
    pl.kernel,
    mesh=plsc.VectorSubcoreMesh(core_axis_name="c", subcore_axis_name="s"),
    out_type=jax.ShapeDtypeStruct((BATCH, SEQ, DIM), jnp.float32),
    scratch_types=[
        pltpu.VMEM((_ICHUNK, SEQ), jnp.int32),
        pltpu.VMEM((_ICHUNK, SEQ), jnp.int32),
        pltpu.VMEM((_ICHUNK, SEQ, DIM), jnp.float32),
        pltpu.VMEM((_ICHUNK, SEQ, DIM), jnp.float32),
        pltpu.SemaphoreType.DMA,
        pltpu.SemaphoreType.DMA,
        pltpu.SemaphoreType.DMA,
        pltpu.SemaphoreType.DMA,
    ],
    compiler_params=pltpu.CompilerParams(use_tc_tiling_on_sc=False),
)
def _gather_kernel(x_hbm, table_hbm, out_hbm, idx0, idx1, rows0, rows1,
                   gsem0, gsem1, ssem0, ssem1):
    wid = lax.axis_index("s") * _NC + lax.axis_index("c")
    ibase = wid * _IPW
    idxs = (idx0, idx1)
    rows = (rows0, rows1)
    gsems = (gsem0, gsem1)
    ssems = (ssem0, ssem1)

    def fire_chunk(c, b):
        i0 = ibase + c * _ICHUNK
        pltpu.sync_copy(x_hbm.at[pl.ds(i0, _ICHUNK)], idxs[b])

        def gather_one(k, carry):
            pltpu.async_copy(table_hbm.at[idxs[b].at[k]], rows[b].at[k],
                             gsems[b])
            return carry

        lax.fori_loop(0, _ICHUNK, gather_one, 0)

    def drain_chunk(c, b):
        i0 = ibase + c * _ICHUNK
        pltpu.make_async_copy(out_hbm.at[pl.ds(i0, _ICHUNK)], rows[b],
                              gsems[b]).wait()

    stores = [None] * _NCHUNK
    for c in range(_NCHUNK):
        b = c % 2
        if c >= 2:
            stores[c - 2].wait()
        fire_chunk(c, b)
        if c >= 1:
            p = 1 - b
            drain_chunk(c - 1, p)
            stores[c - 1] = pltpu.async_copy(
                rows[p],
                out_hbm.at[pl.ds(ibase + (c - 1) * _ICHUNK, _ICHUNK)],
                ssems[p])
    last = _NCHUNK - 1
    drain_chunk(last, last % 2)
    stores[last] = pltpu.async_copy(
        rows[last % 2], out_hbm.at[pl.ds(ibase + last * _ICHUNK, _ICHUNK)],
        ssems[last % 2])
    stores[last - 1].wait()
    stores[last].wait()


def kernel(x, table):
    return _gather_kernel(x.astype(jnp.int32), table)
